# baseline (speedup 1.0000x reference)
import functools

import jax
import jax.numpy as jnp
from jax import lax
from jax.experimental import pallas as pl
from jax.experimental.pallas import tpu as pltpu
from jax.experimental.pallas import tpu_sc as plsc

N_NODES = 10000
N_EDGES = 320000
D = 128

NC = 2
NS = 16
NW = NC * NS
CHUNK = 80
N_CHUNKS = 126
E_PER_W = CHUNK * N_CHUNKS
E_TOTAL = E_PER_W * NW
N_PAD = 10112
ROWS_PER_S = N_PAD // NS


def _sc_aggregate(h, src, dst, zeros):
    mesh = plsc.VectorSubcoreMesh(core_axis_name="c", subcore_axis_name="s")

    @functools.partial(
        pl.kernel,
        out_type=jax.ShapeDtypeStruct((NC, N_PAD, D), jnp.float32),
        mesh=mesh,
        scratch_types=[
            pltpu.VMEM((E_PER_W,), jnp.int32),
            pltpu.VMEM((N_CHUNKS, CHUNK), jnp.int32),
            pltpu.VMEM((2, CHUNK, D), jnp.float32),
            pltpu.VMEM_SHARED((N_PAD, D), jnp.float32),
            pltpu.SemaphoreType.DMA,
        ],
    )
    def agg(h_hbm, src_hbm, dst_hbm, zeros_hbm, out_hbm,
            src_v, dst_v, rows_v, acc, sem):
        c = lax.axis_index("c")
        s = lax.axis_index("s")
        wid = c * NS + s

        pltpu.sync_copy(zeros_hbm.at[pl.ds(s * ROWS_PER_S, ROWS_PER_S)],
                        acc.at[pl.ds(s * ROWS_PER_S, ROWS_PER_S)])
        plsc.subcore_barrier()

        pltpu.sync_copy(src_hbm.at[wid], src_v)
        pltpu.sync_copy(dst_hbm.at[wid], dst_v)

        def gather(j, buf):
            return pltpu.make_async_copy(
                h_hbm.at[src_v.at[pl.ds(j * CHUNK, CHUNK)]],
                rows_v.at[buf], sem)

        def scatter_sync(j, buf):
            pltpu.sync_copy(rows_v.at[buf], acc.at[dst_v.at[j]], add=True)

        gather(0, 0).start()

        def body(k, carry):
            i = 2 * k
            gather(i, 0).wait()
            gather(i + 1, 1).start()
            scatter_sync(i, 0)
            gather(i + 1, 1).wait()

            @pl.when(i + 2 < N_CHUNKS)
            def _():
                gather(i + 2, 0).start()

            scatter_sync(i + 1, 1)
            return carry

        lax.fori_loop(0, N_CHUNKS // 2, body, 0)
        plsc.subcore_barrier()

        pltpu.sync_copy(acc.at[pl.ds(s * ROWS_PER_S, ROWS_PER_S)],
                        out_hbm.at[c, pl.ds(s * ROWS_PER_S, ROWS_PER_S)])

    return agg(h, src, dst, zeros)


def _tc_linear(partials, W, b):
    BLK = 400
    grid = N_NODES // BLK

    def body(p0_ref, p1_ref, w_ref, b_ref, out_ref):
        ah = p0_ref[0] + p1_ref[0]
        out_ref[...] = jnp.maximum(
            jnp.dot(ah, w_ref[...], preferred_element_type=jnp.float32)
            + b_ref[...], 0.0)

    return pl.pallas_call(
        body,
        grid=(grid,),
        in_specs=[
            pl.BlockSpec((1, BLK, D), lambda i: (0, i, 0)),
            pl.BlockSpec((1, BLK, D), lambda i: (1, i, 0)),
            pl.BlockSpec((D, D), lambda i: (0, 0)),
            pl.BlockSpec((1, D), lambda i: (0, 0)),
        ],
        out_specs=pl.BlockSpec((BLK, D), lambda i: (i, 0)),
        out_shape=jax.ShapeDtypeStruct((N_NODES, D), jnp.float32),
    )(partials, partials, W, b)


def kernel(h, edge_index, W, b):
    ei = edge_index.astype(jnp.int32)
    n_dummy = E_TOTAL - N_EDGES
    src = jnp.concatenate([ei[0], jnp.zeros((n_dummy,), jnp.int32)])
    pad_dst = N_NODES + jnp.arange(n_dummy, dtype=jnp.int32) % (N_PAD - N_NODES)
    dst = jnp.concatenate([ei[1], pad_dst])
    src = src.reshape(NW, E_PER_W)
    dst = dst.reshape(NW, N_CHUNKS, CHUNK)
    zeros = jnp.zeros((N_PAD, D), jnp.float32)
    partials = _sc_aggregate(h, src, dst, zeros)
    return _tc_linear(partials, W, b.reshape(1, D))


# speedup vs baseline: 1.6014x; 1.6014x over previous
import functools

import jax
import jax.numpy as jnp
from jax import lax
from jax.experimental import pallas as pl
from jax.experimental.pallas import tpu as pltpu
from jax.experimental.pallas import tpu_sc as plsc

N_NODES = 10000
N_EDGES = 320000
D = 128

NC = 2
NS = 16
NW = NC * NS
E_PER_W = N_EDGES // NW
CHUNK = 80
N_CHUNKS = E_PER_W // CHUNK
N_PAD = 10112
ROWS_PER_S = N_PAD // NS
NROW = 3
NDST = 4


def _sc_aggregate(h, src, dst, zeros):
    mesh = plsc.VectorSubcoreMesh(core_axis_name="c", subcore_axis_name="s")

    @functools.partial(
        pl.kernel,
        out_type=jax.ShapeDtypeStruct((NC, N_PAD, D), jnp.float32),
        mesh=mesh,
        scratch_types=[
            pltpu.VMEM((E_PER_W,), jnp.int32),
            pltpu.VMEM((NDST, 1, CHUNK), jnp.int32),
            pltpu.VMEM((NROW, CHUNK, D), jnp.float32),
            pltpu.VMEM_SHARED((N_PAD, D), jnp.float32),
            pltpu.SemaphoreType.DMA,
            pltpu.SemaphoreType.DMA,
            pltpu.SemaphoreType.DMA,
        ],
    )
    def agg(h_hbm, src_hbm, dst_hbm, zeros_hbm, out_hbm,
            src_v, dst_v, rows_v, acc, sem_d, sem_g, sem_s):
        c = lax.axis_index("c")
        s = lax.axis_index("s")
        wid = c * NS + s

        pltpu.sync_copy(zeros_hbm.at[pl.ds(s * ROWS_PER_S, ROWS_PER_S)],
                        acc.at[pl.ds(s * ROWS_PER_S, ROWS_PER_S)])
        plsc.subcore_barrier()

        pltpu.sync_copy(src_hbm.at[wid], src_v)

        def dst_copy(j):
            return pltpu.make_async_copy(
                dst_hbm.at[wid, j], dst_v.at[lax.rem(j, NDST)], sem_d)

        def gather(j):
            return pltpu.make_async_copy(
                h_hbm.at[src_v.at[pl.ds(j * CHUNK, CHUNK)]],
                rows_v.at[lax.rem(j, NROW)], sem_g)

        def scatter(j):
            return pltpu.make_async_copy(
                rows_v.at[lax.rem(j, NROW)],
                acc.at[dst_v.at[lax.rem(j, NDST), 0]], sem_s)

        pltpu.sync_copy(dst_hbm.at[wid, 0], dst_v.at[0])
        pltpu.sync_copy(dst_hbm.at[wid, 1], dst_v.at[1])
        gather(0).start()

        def body(i, carry):
            @pl.when(i >= 2)
            def _():
                scatter(i - 2).wait()

            @pl.when(i + 2 < N_CHUNKS)
            def _():
                dst_copy(i + 2).start()

            gather(i).wait()

            @pl.when(i + 1 < N_CHUNKS)
            def _():
                gather(i + 1).start()

            @pl.when(i >= 2)
            def _():
                dst_copy(i).wait()

            pltpu.async_copy(rows_v.at[lax.rem(i, NROW)],
                             acc.at[dst_v.at[lax.rem(i, NDST), 0]],
                             sem_s, add=True)
            return carry

        lax.fori_loop(0, N_CHUNKS, body, 0)
        scatter(N_CHUNKS - 2).wait()
        scatter(N_CHUNKS - 1).wait()
        plsc.subcore_barrier()

        pltpu.sync_copy(acc.at[pl.ds(s * ROWS_PER_S, ROWS_PER_S)],
                        out_hbm.at[c, pl.ds(s * ROWS_PER_S, ROWS_PER_S)])

    return agg(h, src, dst, zeros)


def _tc_linear(partials, W, b):
    BLK = 400
    grid = N_NODES // BLK

    def body(p0_ref, p1_ref, w_ref, b_ref, out_ref):
        ah = p0_ref[0] + p1_ref[0]
        out_ref[...] = jnp.maximum(
            jnp.dot(ah, w_ref[...], preferred_element_type=jnp.float32)
            + b_ref[...], 0.0)

    return pl.pallas_call(
        body,
        grid=(grid,),
        in_specs=[
            pl.BlockSpec((1, BLK, D), lambda i: (0, i, 0)),
            pl.BlockSpec((1, BLK, D), lambda i: (1, i, 0)),
            pl.BlockSpec((D, D), lambda i: (0, 0)),
            pl.BlockSpec((1, D), lambda i: (0, 0)),
        ],
        out_specs=pl.BlockSpec((BLK, D), lambda i: (i, 0)),
        out_shape=jax.ShapeDtypeStruct((N_NODES, D), jnp.float32),
    )(partials, partials, W, b)


def kernel(h, edge_index, W, b):
    ei = edge_index.astype(jnp.int32)
    src = ei[0].reshape(NW, E_PER_W)
    dst = ei[1].reshape(NW, N_CHUNKS, 1, CHUNK)
    zeros = jnp.zeros((N_PAD, D), jnp.float32)
    partials = _sc_aggregate(h, src, dst, zeros)
    return _tc_linear(partials, W, b.reshape(1, D))


# speedup vs baseline: 2.3321x; 1.4563x over previous
import functools

import jax
import jax.numpy as jnp
from jax import lax
from jax.experimental import pallas as pl
from jax.experimental.pallas import tpu as pltpu
from jax.experimental.pallas import tpu_sc as plsc

N_NODES = 10000
N_EDGES = 320000
D = 128

NC = 2
NS = 16
NW = NC * NS
E_PER_W = N_EDGES // NW
CHUNK = 80
N_CHUNKS = E_PER_W // CHUNK
N_PAD = 10112
ROWS_PER_S = N_PAD // NS
NROW = 4
NIDX = 6


def _sc_aggregate(h, src, dst, zeros):
    mesh = plsc.VectorSubcoreMesh(core_axis_name="c", subcore_axis_name="s")

    @functools.partial(
        pl.kernel,
        out_type=jax.ShapeDtypeStruct((NC, N_PAD, D), jnp.float32),
        mesh=mesh,
        scratch_types=[
            pltpu.VMEM((NIDX, CHUNK), jnp.int32),
            pltpu.VMEM((NIDX, CHUNK), jnp.int32),
            pltpu.VMEM((NROW, CHUNK, D), jnp.float32),
            pltpu.VMEM_SHARED((N_PAD, D), jnp.float32),
            pltpu.SemaphoreType.DMA,
            pltpu.SemaphoreType.DMA,
            pltpu.SemaphoreType.DMA,
            pltpu.SemaphoreType.DMA,
        ],
    )
    def agg(h_hbm, src_hbm, dst_hbm, zeros_hbm, out_hbm,
            src_v, dst_v, rows_v, acc, sem_cs, sem_cd, sem_g, sem_s):
        c = lax.axis_index("c")
        s = lax.axis_index("s")
        wid = c * NS + s
        base = wid * E_PER_W

        pltpu.sync_copy(zeros_hbm.at[pl.ds(s * ROWS_PER_S, ROWS_PER_S)],
                        acc.at[pl.ds(s * ROWS_PER_S, ROWS_PER_S)])
        plsc.subcore_barrier()

        def copy_s(j):
            return pltpu.make_async_copy(
                src_hbm.at[pl.ds(base + j * CHUNK, CHUNK)],
                src_v.at[lax.rem(j, NIDX)], sem_cs)

        def copy_d(j):
            return pltpu.make_async_copy(
                dst_hbm.at[pl.ds(base + j * CHUNK, CHUNK)],
                dst_v.at[lax.rem(j, NIDX)], sem_cd)

        def gather(j):
            return pltpu.make_async_copy(
                h_hbm.at[src_v.at[lax.rem(j, NIDX)]],
                rows_v.at[lax.rem(j, NROW)], sem_g)

        def scatter(j):
            return pltpu.make_async_copy(
                rows_v.at[lax.rem(j, NROW)],
                acc.at[dst_v.at[lax.rem(j, NIDX)]], sem_s)

        for j in range(5):
            pltpu.sync_copy(src_hbm.at[pl.ds(base + j * CHUNK, CHUNK)],
                            src_v.at[j])
            pltpu.sync_copy(dst_hbm.at[pl.ds(base + j * CHUNK, CHUNK)],
                            dst_v.at[j])
        gather(0).start()
        gather(1).start()
        gather(2).start()

        def body(i, carry):
            @pl.when(i >= 1)
            def _():
                scatter(i - 1).wait()

            @pl.when(i + 5 < N_CHUNKS)
            def _():
                copy_s(i + 5).start()
                copy_d(i + 5).start()

            @pl.when(jnp.logical_and(i + 3 >= 5, i + 3 < N_CHUNKS))
            def _():
                copy_s(i + 3).wait()
                copy_d(i + 3).wait()

            gather(i).wait()

            @pl.when(i + 3 < N_CHUNKS)
            def _():
                gather(i + 3).start()

            pltpu.async_copy(rows_v.at[lax.rem(i, NROW)],
                             acc.at[dst_v.at[lax.rem(i, NIDX)]],
                             sem_s, add=True)
            return carry

        lax.fori_loop(0, N_CHUNKS, body, 0)
        scatter(N_CHUNKS - 1).wait()
        plsc.subcore_barrier()

        pltpu.sync_copy(acc.at[pl.ds(s * ROWS_PER_S, ROWS_PER_S)],
                        out_hbm.at[c, pl.ds(s * ROWS_PER_S, ROWS_PER_S)])

    return agg(h, src, dst, zeros)


def _tc_linear(partials, W, b):
    BLK = 400
    grid = N_NODES // BLK

    def body(p0_ref, p1_ref, w_ref, b_ref, out_ref):
        ah = p0_ref[0] + p1_ref[0]
        out_ref[...] = jnp.maximum(
            jnp.dot(ah, w_ref[...], preferred_element_type=jnp.float32)
            + b_ref[...], 0.0)

    return pl.pallas_call(
        body,
        grid=(grid,),
        in_specs=[
            pl.BlockSpec((1, BLK, D), lambda i: (0, i, 0)),
            pl.BlockSpec((1, BLK, D), lambda i: (1, i, 0)),
            pl.BlockSpec((D, D), lambda i: (0, 0)),
            pl.BlockSpec((1, D), lambda i: (0, 0)),
        ],
        out_specs=pl.BlockSpec((BLK, D), lambda i: (i, 0)),
        out_shape=jax.ShapeDtypeStruct((N_NODES, D), jnp.float32),
    )(partials, partials, W, b)


def kernel(h, edge_index, W, b):
    ei = edge_index.astype(jnp.int32)
    zeros = jnp.zeros((N_PAD, D), jnp.float32)
    partials = _sc_aggregate(h, ei[0], ei[1], zeros)
    return _tc_linear(partials, W, b.reshape(1, D))


# speedup vs baseline: 2.7691x; 1.1874x over previous
import functools

import jax
import jax.numpy as jnp
from jax import lax
from jax.experimental import pallas as pl
from jax.experimental.pallas import tpu as pltpu
from jax.experimental.pallas import tpu_sc as plsc

N_NODES = 10000
N_EDGES = 320000
D = 128

NC = 2
NS = 16
NW = NC * NS
CHUNK = 128
N_CHUNKS = 78
E_MAIN = CHUNK * N_CHUNKS
EXTRA_BASE = NW * E_MAIN
SLAB = 624
TAIL = N_NODES - NS * SLAB
NROW = 3
NIDX = 4


def _sc_aggregate(h, edge_index, zeros):
    mesh = plsc.VectorSubcoreMesh(core_axis_name="c", subcore_axis_name="s")

    @functools.partial(
        pl.kernel,
        out_type=jax.ShapeDtypeStruct((NC, N_NODES, D), jnp.float32),
        mesh=mesh,
        scratch_types=[
            pltpu.VMEM((2 * NIDX, CHUNK), jnp.int32),
            pltpu.VMEM((NROW, CHUNK, D), jnp.float32),
            pltpu.VMEM_SHARED((N_NODES, D), jnp.float32),
            pltpu.SemaphoreType.DMA,
            pltpu.SemaphoreType.DMA,
            pltpu.SemaphoreType.DMA,
            pltpu.SemaphoreType.DMA,
        ],
    )
    def agg(h_hbm, e_hbm, zeros_hbm, out_hbm,
            idx_v, rows_v, acc, sem_cs, sem_cd, sem_g, sem_s):
        c = lax.axis_index("c")
        s = lax.axis_index("s")
        wid = c * NS + s
        base = wid * E_MAIN

        def copy_s(j):
            return pltpu.make_async_copy(
                e_hbm.at[0, pl.ds(base + j * CHUNK, CHUNK)],
                idx_v.at[lax.rem(j, NIDX)], sem_cs)

        def copy_d(j):
            return pltpu.make_async_copy(
                e_hbm.at[1, pl.ds(base + j * CHUNK, CHUNK)],
                idx_v.at[NIDX + lax.rem(j, NIDX)], sem_cd)

        def gather(j):
            return pltpu.make_async_copy(
                h_hbm.at[idx_v.at[lax.rem(j, NIDX)]],
                rows_v.at[lax.rem(j, NROW)], sem_g)

        def scatter(j):
            return pltpu.make_async_copy(
                rows_v.at[lax.rem(j, NROW)],
                acc.at[idx_v.at[NIDX + lax.rem(j, NIDX)]], sem_s)

        for j in range(3):
            pltpu.sync_copy(e_hbm.at[0, pl.ds(base + j * CHUNK, CHUNK)],
                            idx_v.at[j])
            pltpu.sync_copy(e_hbm.at[1, pl.ds(base + j * CHUNK, CHUNK)],
                            idx_v.at[NIDX + j])
        gather(0).start()
        gather(1).start()

        pltpu.sync_copy(zeros_hbm.at[pl.ds(0, SLAB)],
                        acc.at[pl.ds(s * SLAB, SLAB)])

        @pl.when(s == NS - 1)
        def _():
            pltpu.sync_copy(zeros_hbm.at[pl.ds(0, TAIL)],
                            acc.at[pl.ds(NS * SLAB, TAIL)])

        plsc.subcore_barrier()

        def body(i, carry):
            @pl.when(i >= 1)
            def _():
                scatter(i - 1).wait()

            @pl.when(i + 3 < N_CHUNKS)
            def _():
                copy_s(i + 3).start()
                copy_d(i + 3).start()

            @pl.when(jnp.logical_and(i + 2 >= 3, i + 2 < N_CHUNKS))
            def _():
                copy_s(i + 2).wait()
                copy_d(i + 2).wait()

            gather(i).wait()

            @pl.when(i + 2 < N_CHUNKS)
            def _():
                gather(i + 2).start()

            pltpu.async_copy(rows_v.at[lax.rem(i, NROW)],
                             acc.at[idx_v.at[NIDX + lax.rem(i, NIDX)]],
                             sem_s, add=True)
            return carry

        lax.fori_loop(0, N_CHUNKS, body, 0)
        scatter(N_CHUNKS - 1).wait()

        @pl.when(wid < 4)
        def _():
            ebase = EXTRA_BASE + wid * CHUNK
            pltpu.sync_copy(e_hbm.at[0, pl.ds(ebase, CHUNK)], idx_v.at[0])
            pltpu.sync_copy(e_hbm.at[1, pl.ds(ebase, CHUNK)],
                            idx_v.at[NIDX])
            pltpu.async_copy(h_hbm.at[idx_v.at[0]], rows_v.at[0],
                             sem_g).wait()
            pltpu.async_copy(rows_v.at[0], acc.at[idx_v.at[NIDX]],
                             sem_s, add=True)
            scatter(0).wait()

        plsc.subcore_barrier()

        pltpu.sync_copy(acc.at[pl.ds(s * SLAB, SLAB)],
                        out_hbm.at[c, pl.ds(s * SLAB, SLAB)])

        @pl.when(s == NS - 1)
        def _():
            pltpu.sync_copy(acc.at[pl.ds(NS * SLAB, TAIL)],
                            out_hbm.at[c, pl.ds(NS * SLAB, TAIL)])

    return agg(h, edge_index, zeros)


def _tc_linear(partials, W, b):
    BLK = 2000
    grid = N_NODES // BLK

    def body(p0_ref, p1_ref, w_ref, b_ref, out_ref):
        ah = p0_ref[0] + p1_ref[0]
        out_ref[...] = jnp.maximum(
            jnp.dot(ah, w_ref[...], preferred_element_type=jnp.float32)
            + b_ref[...], 0.0)

    return pl.pallas_call(
        body,
        grid=(grid,),
        in_specs=[
            pl.BlockSpec((1, BLK, D), lambda i: (0, i, 0)),
            pl.BlockSpec((1, BLK, D), lambda i: (1, i, 0)),
            pl.BlockSpec((D, D), lambda i: (0, 0)),
            pl.BlockSpec((1, D), lambda i: (0, 0)),
        ],
        out_specs=pl.BlockSpec((BLK, D), lambda i: (i, 0)),
        out_shape=jax.ShapeDtypeStruct((N_NODES, D), jnp.float32),
    )(partials, partials, W, b)


def kernel(h, edge_index, W, b):
    ei = edge_index.astype(jnp.int32)
    zeros = jnp.zeros((SLAB, D), jnp.float32)
    partials = _sc_aggregate(h, ei, zeros)
    return _tc_linear(partials, W, b.reshape(1, D))


# speedup vs baseline: 2.7743x; 1.0019x over previous
import functools

import jax
import jax.numpy as jnp
from jax import lax
from jax.experimental import pallas as pl
from jax.experimental.pallas import tpu as pltpu
from jax.experimental.pallas import tpu_sc as plsc

N_NODES = 10000
N_EDGES = 320000
D = 128

NC = 2
NS = 16
NW = NC * NS
CHUNK = 128
N_CHUNKS = 78
E_MAIN = CHUNK * N_CHUNKS
EXTRA_BASE = NW * E_MAIN
SLAB = 624
TAIL = N_NODES - NS * SLAB
NROW = 3
NIDX = 4


def _sc_aggregate(h, edge_index, zeros):
    mesh = plsc.VectorSubcoreMesh(core_axis_name="c", subcore_axis_name="s")

    @functools.partial(
        pl.kernel,
        out_type=jax.ShapeDtypeStruct((NC, N_NODES, D), jnp.float32),
        mesh=mesh,
        scratch_types=[
            pltpu.VMEM((2 * NIDX, CHUNK), jnp.int32),
            pltpu.VMEM((NROW, CHUNK, D), jnp.float32),
            pltpu.VMEM_SHARED((N_NODES, D), jnp.float32),
            pltpu.SemaphoreType.DMA,
            pltpu.SemaphoreType.DMA,
            pltpu.SemaphoreType.DMA,
            pltpu.SemaphoreType.DMA,
        ],
    )
    def agg(h_hbm, e_hbm, zeros_hbm, out_hbm,
            idx_v, rows_v, acc, sem_cs, sem_cd, sem_g, sem_s):
        c = lax.axis_index("c")
        s = lax.axis_index("s")
        wid = c * NS + s
        base = wid * E_MAIN

        def copy_s(j):
            return pltpu.make_async_copy(
                e_hbm.at[0, pl.ds(base + j * CHUNK, CHUNK)],
                idx_v.at[lax.rem(j, NIDX)], sem_cs)

        def copy_d(j):
            return pltpu.make_async_copy(
                e_hbm.at[1, pl.ds(base + j * CHUNK, CHUNK)],
                idx_v.at[NIDX + lax.rem(j, NIDX)], sem_cd)

        def gather(j):
            return pltpu.make_async_copy(
                h_hbm.at[idx_v.at[lax.rem(j, NIDX)]],
                rows_v.at[lax.rem(j, NROW)], sem_g)

        def scatter(j):
            return pltpu.make_async_copy(
                rows_v.at[lax.rem(j, NROW)],
                acc.at[idx_v.at[NIDX + lax.rem(j, NIDX)]], sem_s)

        for j in range(3):
            pltpu.sync_copy(e_hbm.at[0, pl.ds(base + j * CHUNK, CHUNK)],
                            idx_v.at[j])
            pltpu.sync_copy(e_hbm.at[1, pl.ds(base + j * CHUNK, CHUNK)],
                            idx_v.at[NIDX + j])
        gather(0).start()
        gather(1).start()

        pltpu.sync_copy(zeros_hbm.at[pl.ds(0, SLAB)],
                        acc.at[pl.ds(s * SLAB, SLAB)])

        @pl.when(s == NS - 1)
        def _():
            pltpu.sync_copy(zeros_hbm.at[pl.ds(0, TAIL)],
                            acc.at[pl.ds(NS * SLAB, TAIL)])

        plsc.subcore_barrier()

        def body(i, carry):
            @pl.when(i >= 1)
            def _():
                scatter(i - 1).wait()

            @pl.when(i + 3 < N_CHUNKS)
            def _():
                copy_s(i + 3).start()
                copy_d(i + 3).start()

            @pl.when(jnp.logical_and(i + 2 >= 3, i + 2 < N_CHUNKS))
            def _():
                copy_s(i + 2).wait()
                copy_d(i + 2).wait()

            gather(i).wait()

            @pl.when(i + 2 < N_CHUNKS)
            def _():
                gather(i + 2).start()

            pltpu.async_copy(rows_v.at[lax.rem(i, NROW)],
                             acc.at[idx_v.at[NIDX + lax.rem(i, NIDX)]],
                             sem_s, add=True)
            return carry

        lax.fori_loop(0, N_CHUNKS, body, 0)
        scatter(N_CHUNKS - 1).wait()

        @pl.when(s < 2)
        def _():
            ebase = EXTRA_BASE + (c * 2 + s) * CHUNK
            pltpu.sync_copy(e_hbm.at[0, pl.ds(ebase, CHUNK)], idx_v.at[0])
            pltpu.sync_copy(e_hbm.at[1, pl.ds(ebase, CHUNK)],
                            idx_v.at[NIDX])
            pltpu.async_copy(h_hbm.at[idx_v.at[0]], rows_v.at[0],
                             sem_g).wait()
            pltpu.async_copy(rows_v.at[0], acc.at[idx_v.at[NIDX]],
                             sem_s, add=True)
            scatter(0).wait()

        plsc.subcore_barrier()

        pltpu.sync_copy(acc.at[pl.ds(s * SLAB, SLAB)],
                        out_hbm.at[c, pl.ds(s * SLAB, SLAB)])

        @pl.when(s == NS - 1)
        def _():
            pltpu.sync_copy(acc.at[pl.ds(NS * SLAB, TAIL)],
                            out_hbm.at[c, pl.ds(NS * SLAB, TAIL)])

    return agg(h, edge_index, zeros)


def _tc_linear(partials, W, b):
    BLK = 2000
    grid = N_NODES // BLK

    def body(p0_ref, p1_ref, w_ref, b_ref, out_ref):
        ah = p0_ref[0] + p1_ref[0]
        out_ref[...] = jnp.maximum(
            jnp.dot(ah, w_ref[...], preferred_element_type=jnp.float32)
            + b_ref[...], 0.0)

    return pl.pallas_call(
        body,
        grid=(grid,),
        in_specs=[
            pl.BlockSpec((1, BLK, D), lambda i: (0, i, 0)),
            pl.BlockSpec((1, BLK, D), lambda i: (1, i, 0)),
            pl.BlockSpec((D, D), lambda i: (0, 0)),
            pl.BlockSpec((1, D), lambda i: (0, 0)),
        ],
        out_specs=pl.BlockSpec((BLK, D), lambda i: (i, 0)),
        out_shape=jax.ShapeDtypeStruct((N_NODES, D), jnp.float32),
    )(partials, partials, W, b)


def kernel(h, edge_index, W, b):
    ei = edge_index.astype(jnp.int32)
    zeros = jnp.zeros((SLAB, D), jnp.float32)
    partials = _sc_aggregate(h, ei, zeros)
    return _tc_linear(partials, W, b.reshape(1, D))
